# baseline (speedup 1.0000x reference)
import functools

import jax
import jax.numpy as jnp
from jax import lax
from jax.experimental import pallas as pl
from jax.experimental.pallas import tpu as pltpu
from jax.experimental.pallas import tpu_sc as plsc


def _sc_gather(table, idx):
    V, D = table.shape
    Btot = idx.shape[0]
    info = plsc.get_sparse_core_info()
    NC_, NS, L = info.num_cores, info.num_subcores, info.num_lanes
    NW = NC_ * NS
    assert D % L == 0 and Btot % (8 * NW) == 0
    b_per_w = Btot // NW
    CH = 512
    while b_per_w % CH != 0:
        CH //= 2
    n_chunks = b_per_w // CH
    mesh = plsc.VectorSubcoreMesh(core_axis_name="c", subcore_axis_name="s")

    @functools.partial(
        pl.kernel, mesh=mesh,
        compiler_params=pltpu.CompilerParams(use_tc_tiling_on_sc=False),
        out_type=jax.ShapeDtypeStruct((Btot, D), table.dtype),
        scratch_types=[
            pltpu.VMEM((b_per_w,), jnp.int32),
            pltpu.VMEM((CH, D), table.dtype),
            pltpu.VMEM((CH, D), table.dtype),
            pltpu.SemaphoreType.DMA,
            pltpu.SemaphoreType.DMA,
            pltpu.SemaphoreType.DMA,
            pltpu.SemaphoreType.DMA,
        ],
    )
    def k(table_hbm, idx_hbm, out_hbm, idx_v, buf0, buf1, sg0, sg1, sw0, sw1):
        wid = lax.axis_index("s") * NC_ + lax.axis_index("c")
        base = wid * b_per_w
        pltpu.sync_copy(idx_hbm.at[pl.ds(base, b_per_w)], idx_v)
        bufs = (buf0, buf1)
        gsems = (sg0, sg1)
        wsems = (sw0, sw1)

        def gather_start(g, slot):
            pltpu.async_copy(
                table_hbm.at[idx_v.at[pl.ds(g * CH, CH)]], bufs[slot], gsems[slot])

        def write_start(g, slot):
            pltpu.async_copy(
                bufs[slot], out_hbm.at[pl.ds(base + g * CH, CH)], wsems[slot])

        gather_start(0, 0)

        def body(i, carry):
            for b in (0, 1):
                g = 2 * i + b
                nslot = 1 - b
                pltpu.make_async_copy(
                    table_hbm.at[idx_v.at[pl.ds(g * CH, CH)]], bufs[b], gsems[b]
                ).wait()

                @pl.when(g + 1 < n_chunks)
                def _():
                    @pl.when(g >= 1)
                    def _():
                        pltpu.make_async_copy(
                            bufs[nslot],
                            out_hbm.at[pl.ds(base + (g - 1) * CH, CH)],
                            wsems[nslot],
                        ).wait()
                    gather_start(g + 1, nslot)

                write_start(g, b)
            return carry

        lax.fori_loop(0, n_chunks // 2, body, 0)
        pltpu.make_async_copy(
            bufs[(n_chunks - 1) % 2],
            out_hbm.at[pl.ds(base + (n_chunks - 1) * CH, CH)],
            wsems[(n_chunks - 1) % 2],
        ).wait()

        @pl.when(n_chunks >= 2)
        def _():
            pltpu.make_async_copy(
                bufs[(n_chunks - 2) % 2],
                out_hbm.at[pl.ds(base + (n_chunks - 2) * CH, CH)],
                wsems[(n_chunks - 2) % 2],
            ).wait()

    return k(table, idx)


def _tc_body(g_ref, next_ref, a1_ref, c1_ref, w2_ref, b2_ref, w3_ref, b3_ref,
             wf_ref, bias_ref, rep_ref, out_ref, v_ref, *, mt, kk, c_in):
    D_PAD = 16
    G = g_ref[...]
    feat = G[:, :c_in]
    ptsp = G[:, c_in:c_in + D_PAD].astype(jnp.float32)

    a1 = a1_ref[...]
    r = jnp.dot(next_ref[...], a1, preferred_element_type=jnp.float32)
    h1 = jnp.dot(ptsp, a1, preferred_element_type=jnp.float32)
    h1 = h1.reshape(mt, kk, 32) - r[:, None, :] + c1_ref[...][None, None, :]
    h1 = jnp.maximum(h1.reshape(mt * kk, 32), 0.0)
    h2 = jnp.maximum(
        jnp.dot(h1, w2_ref[...], preferred_element_type=jnp.float32)
        + b2_ref[...][None, :], 0.0)
    d = jnp.maximum(
        jnp.dot(h2, w3_ref[...], preferred_element_type=jnp.float32)
        + b3_ref[...][None, :], 0.0)

    PG = 16
    rows = mt * kk
    bd_w = PG * 16
    dt = jnp.dot(d.astype(jnp.bfloat16), rep_ref[...],
                 preferred_element_type=jnp.float32)
    ri = lax.broadcasted_iota(jnp.int32, (rows, bd_w), 0)
    li = lax.broadcasted_iota(jnp.int32, (rows, bd_w), 1)
    msk = (li % PG) == ((ri // kk) % PG)
    dbd = jnp.where(msk, dt, 0.0).astype(jnp.bfloat16)
    grp = PG * kk
    for g in range(mt // PG):
        dg = lax.slice(dbd, (g * grp, 0), ((g + 1) * grp, bd_w))
        fg = lax.slice(feat, (g * grp, 0), ((g + 1) * grp, c_in))
        vg = lax.dot_general(
            dg, fg, (((0,), (0,)), ((), ())),
            preferred_element_type=jnp.float32)
        for n in range(16):
            v_ref[pl.ds(g * PG, PG), pl.ds(n * c_in, c_in)] = (
                lax.slice(vg, (n * PG, 0), ((n + 1) * PG, c_in)))
    out_ref[...] = (
        jnp.dot(v_ref[...], wf_ref[...], preferred_element_type=jnp.float32)
        + bias_ref[...][None, :])


def _tc_compute(gathered, next_pad, a1_pad, c1, w2t, b2, w3t, b3, wflat, bias,
                rep, mt, kk, c_in, c_out):
    G = gathered.shape[0] // kk
    n_tiles = G // mt
    grid = (n_tiles,)
    body = functools.partial(_tc_body, mt=mt, kk=kk, c_in=c_in)
    return pl.pallas_call(
        body,
        grid=grid,
        in_specs=[
            pl.BlockSpec((mt * kk, c_in + 32), lambda i: (i, 0)),
            pl.BlockSpec((mt, 16), lambda i: (i, 0)),
            pl.BlockSpec((16, 32), lambda i: (0, 0)),
            pl.BlockSpec((32,), lambda i: (0,)),
            pl.BlockSpec((32, 16), lambda i: (0, 0)),
            pl.BlockSpec((16,), lambda i: (0,)),
            pl.BlockSpec((16, 16), lambda i: (0, 0)),
            pl.BlockSpec((16,), lambda i: (0,)),
            pl.BlockSpec((16 * c_in, c_out), lambda i: (0, 0)),
            pl.BlockSpec((c_out,), lambda i: (0,)),
            pl.BlockSpec((16, 256), lambda i: (0, 0)),
        ],
        out_specs=pl.BlockSpec((mt, c_out), lambda i: (i, 0)),
        out_shape=jax.ShapeDtypeStruct((G, c_out), jnp.float32),
        scratch_shapes=[pltpu.VMEM((mt, 16 * c_in), jnp.float32)],
    )(gathered, next_pad, a1_pad, c1, w2t, b2, w3t, b3, wflat, bias, rep)


def kernel(inp, points, next_pts, indices_, K, weight, bias, centers,
           w1, b1, w2, b2, w3, b3):
    B, N, C_IN = inp.shape
    KK = indices_.shape[2]
    DIM = points.shape[2]
    NC = centers.shape[1]
    C_OUT = weight.shape[2]

    inp_flat = inp.reshape(B * N, C_IN)
    pts_pad = jnp.pad(points.reshape(B * N, DIM), ((0, 0), (0, 16 - DIM)))
    table = jnp.concatenate([inp_flat, pts_pad], axis=1)
    table = table.astype(jnp.bfloat16).reshape(B * N, 40, 2)
    table = jax.lax.bitcast_convert_type(table, jnp.int32)
    table = jnp.pad(table, ((0, 0), (0, 8)))
    offs = (jnp.arange(B, dtype=jnp.int32) * N)[:, None, None]
    idx = (indices_.astype(jnp.int32) + offs).reshape(-1)

    next_pad = jnp.pad(next_pts.reshape(B * N, DIM), ((0, 0), (0, 16 - DIM)))

    a1 = w1.reshape(2 * NC, DIM, NC).sum(axis=2).T
    a1_pad = jnp.pad(a1, ((0, 16 - DIM), (0, 0)))
    c1 = b1 - w1 @ centers.reshape(DIM * NC)
    wflat = weight.transpose(1, 0, 2).reshape(NC * C_IN, C_OUT) / KK
    rep = (jnp.arange(256)[None, :] // 16 ==
           jnp.arange(16)[:, None]).astype(jnp.bfloat16)

    gathered = _sc_gather(table, idx)
    gath_bf = jax.lax.bitcast_convert_type(
        gathered, jnp.bfloat16).reshape(B * N * KK, 96)
    out = _tc_compute(gath_bf, next_pad, a1_pad, c1, w2.T, b2, w3.T, b3,
                      wflat, bias, rep, mt=256, kk=KK, c_in=C_IN, c_out=C_OUT)
    return out.reshape(B, N, C_OUT)


# speedup vs baseline: 2.1792x; 2.1792x over previous
import functools

import jax
import jax.numpy as jnp
from jax import lax
from jax.experimental import pallas as pl
from jax.experimental.pallas import tpu as pltpu
from jax.experimental.pallas import tpu_sc as plsc


def _sc_gather(table, idx):
    V, D = table.shape
    Btot = idx.shape[0]
    info = plsc.get_sparse_core_info()
    NC_, NS, L = info.num_cores, info.num_subcores, info.num_lanes
    NW = NC_ * NS
    assert D % L == 0 and Btot % (8 * NW) == 0
    b_per_w = Btot // NW
    CH = 512
    while b_per_w % CH != 0:
        CH //= 2
    n_chunks = b_per_w // CH
    mesh = plsc.VectorSubcoreMesh(core_axis_name="c", subcore_axis_name="s")

    @functools.partial(
        pl.kernel, mesh=mesh,
        compiler_params=pltpu.CompilerParams(use_tc_tiling_on_sc=False),
        out_type=jax.ShapeDtypeStruct((Btot, D), table.dtype),
        scratch_types=[
            pltpu.VMEM((b_per_w,), jnp.int32),
            pltpu.VMEM((CH, D), table.dtype),
            pltpu.VMEM((CH, D), table.dtype),
            pltpu.SemaphoreType.DMA,
            pltpu.SemaphoreType.DMA,
            pltpu.SemaphoreType.DMA,
            pltpu.SemaphoreType.DMA,
        ],
    )
    def k(table_hbm, idx_hbm, out_hbm, idx_v, buf0, buf1, sg0, sg1, sw0, sw1):
        wid = lax.axis_index("s") * NC_ + lax.axis_index("c")
        base = wid * b_per_w
        pltpu.sync_copy(idx_hbm.at[pl.ds(base, b_per_w)], idx_v)
        bufs = (buf0, buf1)
        gsems = (sg0, sg1)
        wsems = (sw0, sw1)

        def gather_start(g, slot):
            pltpu.async_copy(
                table_hbm.at[idx_v.at[pl.ds(g * CH, CH)]], bufs[slot], gsems[slot])

        def write_start(g, slot):
            pltpu.async_copy(
                bufs[slot], out_hbm.at[pl.ds(base + g * CH, CH)], wsems[slot])

        gather_start(0, 0)

        def body(i, carry):
            for b in (0, 1):
                g = 2 * i + b
                nslot = 1 - b
                pltpu.make_async_copy(
                    table_hbm.at[idx_v.at[pl.ds(g * CH, CH)]], bufs[b], gsems[b]
                ).wait()

                @pl.when(g + 1 < n_chunks)
                def _():
                    @pl.when(g >= 1)
                    def _():
                        pltpu.make_async_copy(
                            bufs[nslot],
                            out_hbm.at[pl.ds(base + (g - 1) * CH, CH)],
                            wsems[nslot],
                        ).wait()
                    gather_start(g + 1, nslot)

                write_start(g, b)
            return carry

        lax.fori_loop(0, n_chunks // 2, body, 0)
        pltpu.make_async_copy(
            bufs[(n_chunks - 1) % 2],
            out_hbm.at[pl.ds(base + (n_chunks - 1) * CH, CH)],
            wsems[(n_chunks - 1) % 2],
        ).wait()

        @pl.when(n_chunks >= 2)
        def _():
            pltpu.make_async_copy(
                bufs[(n_chunks - 2) % 2],
                out_hbm.at[pl.ds(base + (n_chunks - 2) * CH, CH)],
                wsems[(n_chunks - 2) % 2],
            ).wait()

    return k(table, idx)


def _tc_body(g_ref, next_ref, a1_ref, c1_ref, w2_ref, b2_ref, w3_ref, b3_ref,
             wf_ref, bias_ref, rep_ref, out_ref, v_ref, *, mt, kk, c_in):
    G = g_ref[...]
    lo = lax.bitcast_convert_type(
        lax.shift_left(G, 16), jnp.float32)
    hi = lax.bitcast_convert_type(
        lax.bitwise_and(G, jnp.int32(-65536)), jnp.float32)
    half = c_in // 2
    feat = jnp.concatenate([lo[:, :half], hi[:, :half]], axis=1)
    ptsp = jnp.concatenate([lo[:, half:half + 8], hi[:, half:half + 8]], axis=1)

    a1 = a1_ref[...]
    r = jnp.dot(next_ref[...], a1, preferred_element_type=jnp.float32)
    h1 = jnp.dot(ptsp, a1, preferred_element_type=jnp.float32)
    h1 = h1.reshape(mt, kk, 32) - r[:, None, :] + c1_ref[...][None, None, :]
    h1 = jnp.maximum(h1.reshape(mt * kk, 32), 0.0)
    h2 = jnp.maximum(
        jnp.dot(h1, w2_ref[...], preferred_element_type=jnp.float32)
        + b2_ref[...][None, :], 0.0)
    d = jnp.maximum(
        jnp.dot(h2, w3_ref[...], preferred_element_type=jnp.float32)
        + b3_ref[...][None, :], 0.0)

    PG = 16
    rows = mt * kk
    bd_w = PG * 16
    dt = jnp.dot(d.astype(jnp.bfloat16), rep_ref[...],
                 preferred_element_type=jnp.float32)
    ri = lax.broadcasted_iota(jnp.int32, (rows, bd_w), 0)
    li = lax.broadcasted_iota(jnp.int32, (rows, bd_w), 1)
    msk = (li % PG) == ((ri // kk) % PG)
    dbd = jnp.where(msk, dt, 0.0).astype(jnp.bfloat16)
    featb = feat.astype(jnp.bfloat16)
    grp = PG * kk
    for g in range(mt // PG):
        dg = lax.slice(dbd, (g * grp, 0), ((g + 1) * grp, bd_w))
        fg = lax.slice(featb, (g * grp, 0), ((g + 1) * grp, c_in))
        vg = lax.dot_general(
            dg, fg, (((0,), (0,)), ((), ())),
            preferred_element_type=jnp.float32)
        for n in range(16):
            v_ref[pl.ds(g * PG, PG), pl.ds(n * c_in, c_in)] = (
                lax.slice(vg, (n * PG, 0), ((n + 1) * PG, c_in)))
    out_ref[...] = (
        jnp.dot(v_ref[...], wf_ref[...], preferred_element_type=jnp.float32)
        + bias_ref[...][None, :])


def _tc_compute(gathered, next_pad, a1_pad, c1, w2t, b2, w3t, b3, wflat, bias,
                rep, mt, kk, c_in, c_out):
    G = gathered.shape[0] // kk
    n_tiles = G // mt
    grid = (n_tiles,)
    body = functools.partial(_tc_body, mt=mt, kk=kk, c_in=c_in)
    return pl.pallas_call(
        body,
        grid=grid,
        in_specs=[
            pl.BlockSpec((mt * kk, c_in // 2 + 16), lambda i: (i, 0)),
            pl.BlockSpec((mt, 16), lambda i: (i, 0)),
            pl.BlockSpec((16, 32), lambda i: (0, 0)),
            pl.BlockSpec((32,), lambda i: (0,)),
            pl.BlockSpec((32, 16), lambda i: (0, 0)),
            pl.BlockSpec((16,), lambda i: (0,)),
            pl.BlockSpec((16, 16), lambda i: (0, 0)),
            pl.BlockSpec((16,), lambda i: (0,)),
            pl.BlockSpec((16 * c_in, c_out), lambda i: (0, 0)),
            pl.BlockSpec((c_out,), lambda i: (0,)),
            pl.BlockSpec((16, 256), lambda i: (0, 0)),
        ],
        out_specs=pl.BlockSpec((mt, c_out), lambda i: (i, 0)),
        out_shape=jax.ShapeDtypeStruct((G, c_out), jnp.float32),
        scratch_shapes=[pltpu.VMEM((mt, 16 * c_in), jnp.float32)],
    )(gathered, next_pad, a1_pad, c1, w2t, b2, w3t, b3, wflat, bias, rep)


def kernel(inp, points, next_pts, indices_, K, weight, bias, centers,
           w1, b1, w2, b2, w3, b3):
    B, N, C_IN = inp.shape
    KK = indices_.shape[2]
    DIM = points.shape[2]
    NC = centers.shape[1]
    C_OUT = weight.shape[2]

    inp_flat = inp.reshape(B * N, C_IN)
    pts_pad = jnp.pad(points.reshape(B * N, DIM), ((0, 0), (0, 16 - DIM)))
    table = jnp.concatenate([inp_flat, pts_pad], axis=1)
    table = table.astype(jnp.bfloat16).reshape(B * N, 40, 2)
    table = jax.lax.bitcast_convert_type(table, jnp.int32)
    table = jnp.pad(table, ((0, 0), (0, 8)))
    offs = (jnp.arange(B, dtype=jnp.int32) * N)[:, None, None]
    idx = (indices_.astype(jnp.int32) + offs).reshape(-1)

    next_pad = jnp.pad(next_pts.reshape(B * N, DIM), ((0, 0), (0, 16 - DIM)))

    a1 = w1.reshape(2 * NC, DIM, NC).sum(axis=2).T
    a1_pad = jnp.pad(a1, ((0, 16 - DIM), (0, 0)))
    c1 = b1 - w1 @ centers.reshape(DIM * NC)
    perm_c = jnp.concatenate([jnp.arange(0, C_IN, 2), jnp.arange(1, C_IN, 2)])
    perm_d = jnp.concatenate([jnp.arange(0, 16, 2), jnp.arange(1, 16, 2)])
    a1_pad = a1_pad[perm_d, :]
    next_pad = next_pad[:, perm_d]
    wflat = (weight[perm_c, :, :].transpose(1, 0, 2)
             .reshape(NC * C_IN, C_OUT) / KK)
    rep = (jnp.arange(256)[None, :] // 16 ==
           jnp.arange(16)[:, None]).astype(jnp.bfloat16)

    gathered = _sc_gather(table, idx)
    out = _tc_compute(gathered, next_pad, a1_pad, c1, w2.T, b2, w3.T, b3,
                      wflat, bias, rep, mt=256, kk=KK, c_in=C_IN, c_out=C_OUT)
    return out.reshape(B, N, C_OUT)


# speedup vs baseline: 2.7555x; 1.2645x over previous
import functools

import jax
import jax.numpy as jnp
from jax import lax
from jax.experimental import pallas as pl
from jax.experimental.pallas import tpu as pltpu
from jax.experimental.pallas import tpu_sc as plsc


def _sc_gather(table, idx):
    V, D = table.shape
    Btot = idx.shape[0]
    info = plsc.get_sparse_core_info()
    NC_, NS, L = info.num_cores, info.num_subcores, info.num_lanes
    NW = NC_ * NS
    assert D % L == 0 and Btot % (8 * NW) == 0
    b_per_w = Btot // NW
    CH = 256
    while b_per_w % CH != 0:
        CH //= 2
    n_chunks = b_per_w // CH
    mesh = plsc.VectorSubcoreMesh(core_axis_name="c", subcore_axis_name="s")

    @functools.partial(
        pl.kernel, mesh=mesh,
        compiler_params=pltpu.CompilerParams(use_tc_tiling_on_sc=False),
        out_type=jax.ShapeDtypeStruct((Btot, D), table.dtype),
        scratch_types=[
            pltpu.VMEM((b_per_w,), jnp.int32),
            pltpu.VMEM((CH, D), table.dtype),
            pltpu.VMEM((CH, D), table.dtype),
            pltpu.SemaphoreType.DMA,
            pltpu.SemaphoreType.DMA,
            pltpu.SemaphoreType.DMA,
            pltpu.SemaphoreType.DMA,
        ],
    )
    def k(table_hbm, idx_hbm, out_hbm, idx_v, buf0, buf1, sg0, sg1, sw0, sw1):
        wid = lax.axis_index("s") * NC_ + lax.axis_index("c")
        base = wid * b_per_w
        pltpu.sync_copy(idx_hbm.at[pl.ds(base, b_per_w)], idx_v)
        bufs = (buf0, buf1)
        gsems = (sg0, sg1)
        wsems = (sw0, sw1)

        def gather_start(g, slot):
            pltpu.async_copy(
                table_hbm.at[idx_v.at[pl.ds(g * CH, CH)]], bufs[slot], gsems[slot])

        def write_start(g, slot):
            pltpu.async_copy(
                bufs[slot], out_hbm.at[pl.ds(base + g * CH, CH)], wsems[slot])

        gather_start(0, 0)

        def body(i, carry):
            for b in (0, 1):
                g = 2 * i + b
                nslot = 1 - b
                pltpu.make_async_copy(
                    table_hbm.at[idx_v.at[pl.ds(g * CH, CH)]], bufs[b], gsems[b]
                ).wait()

                @pl.when(g + 1 < n_chunks)
                def _():
                    @pl.when(g >= 1)
                    def _():
                        pltpu.make_async_copy(
                            bufs[nslot],
                            out_hbm.at[pl.ds(base + (g - 1) * CH, CH)],
                            wsems[nslot],
                        ).wait()
                    gather_start(g + 1, nslot)

                write_start(g, b)
            return carry

        lax.fori_loop(0, n_chunks // 2, body, 0)
        pltpu.make_async_copy(
            bufs[(n_chunks - 1) % 2],
            out_hbm.at[pl.ds(base + (n_chunks - 1) * CH, CH)],
            wsems[(n_chunks - 1) % 2],
        ).wait()

        @pl.when(n_chunks >= 2)
        def _():
            pltpu.make_async_copy(
                bufs[(n_chunks - 2) % 2],
                out_hbm.at[pl.ds(base + (n_chunks - 2) * CH, CH)],
                wsems[(n_chunks - 2) % 2],
            ).wait()

    return k(table, idx)


def _tc_body(g_ref, next_ref, a1_ref, c1_ref, w2_ref, b2_ref, w3_ref, b3_ref,
             wf_ref, bias_ref, rep_ref, out_ref, v_ref, *, mt, kk, c_in):
    G = g_ref[...]
    feat = G[:, :c_in]
    ptsp = G[:, c_in:c_in + 16]

    a1 = a1_ref[...]
    r = jnp.dot(next_ref[...], a1, preferred_element_type=jnp.float32)
    h1 = jnp.dot(ptsp, a1, preferred_element_type=jnp.float32)
    h1 = h1.reshape(mt, kk, 32) - r[:, None, :] + c1_ref[...][None, None, :]
    h1 = jnp.maximum(h1.reshape(mt * kk, 32), 0.0)
    h2 = jnp.maximum(
        jnp.dot(h1, w2_ref[...], preferred_element_type=jnp.float32)
        + b2_ref[...][None, :], 0.0)
    d = jnp.maximum(
        jnp.dot(h2, w3_ref[...], preferred_element_type=jnp.float32)
        + b3_ref[...][None, :], 0.0)

    PG = 16
    rows = mt * kk
    bd_w = PG * 16
    dt = jnp.dot(d.astype(jnp.bfloat16), rep_ref[...],
                 preferred_element_type=jnp.float32)
    ri = lax.broadcasted_iota(jnp.int32, (rows, bd_w), 0)
    li = lax.broadcasted_iota(jnp.int32, (rows, bd_w), 1)
    msk = (li % PG) == ((ri // kk) % PG)
    dbd = jnp.where(msk, dt, 0.0).astype(jnp.bfloat16)
    featb = feat.astype(jnp.bfloat16)
    grp = PG * kk
    for g in range(mt // PG):
        dg = lax.slice(dbd, (g * grp, 0), ((g + 1) * grp, bd_w))
        fg = lax.slice(featb, (g * grp, 0), ((g + 1) * grp, c_in))
        vg = lax.dot_general(
            dg, fg, (((0,), (0,)), ((), ())),
            preferred_element_type=jnp.float32)
        for n in range(16):
            v_ref[pl.ds(g * PG, PG), pl.ds(n * c_in, c_in)] = (
                lax.slice(vg, (n * PG, 0), ((n + 1) * PG, c_in)))
    out_ref[...] = (
        jnp.dot(v_ref[...], wf_ref[...], preferred_element_type=jnp.float32)
        + bias_ref[...][None, :])


def _tc_compute(gathered, next_pad, a1_pad, c1, w2t, b2, w3t, b3, wflat, bias,
                rep, mt, kk, c_in, c_out):
    G = gathered.shape[0] // kk
    n_tiles = G // mt
    grid = (n_tiles,)
    body = functools.partial(_tc_body, mt=mt, kk=kk, c_in=c_in)
    return pl.pallas_call(
        body,
        grid=grid,
        in_specs=[
            pl.BlockSpec((mt * kk, 128), lambda i: (i, 0)),
            pl.BlockSpec((mt, 16), lambda i: (i, 0)),
            pl.BlockSpec((16, 32), lambda i: (0, 0)),
            pl.BlockSpec((32,), lambda i: (0,)),
            pl.BlockSpec((32, 16), lambda i: (0, 0)),
            pl.BlockSpec((16,), lambda i: (0,)),
            pl.BlockSpec((16, 16), lambda i: (0, 0)),
            pl.BlockSpec((16,), lambda i: (0,)),
            pl.BlockSpec((16 * c_in, c_out), lambda i: (0, 0)),
            pl.BlockSpec((c_out,), lambda i: (0,)),
            pl.BlockSpec((16, 256), lambda i: (0, 0)),
        ],
        out_specs=pl.BlockSpec((mt, c_out), lambda i: (i, 0)),
        out_shape=jax.ShapeDtypeStruct((G, c_out), jnp.float32),
        scratch_shapes=[pltpu.VMEM((mt, 16 * c_in), jnp.float32)],
    )(gathered, next_pad, a1_pad, c1, w2t, b2, w3t, b3, wflat, bias, rep)


def kernel(inp, points, next_pts, indices_, K, weight, bias, centers,
           w1, b1, w2, b2, w3, b3):
    B, N, C_IN = inp.shape
    KK = indices_.shape[2]
    DIM = points.shape[2]
    NC = centers.shape[1]
    C_OUT = weight.shape[2]

    inp_flat = inp.reshape(B * N, C_IN)
    pts_pad = jnp.pad(points.reshape(B * N, DIM), ((0, 0), (0, 16 - DIM)))
    table = jnp.concatenate([inp_flat, pts_pad], axis=1)
    table = jnp.pad(table, ((0, 0), (0, 48)))
    offs = (jnp.arange(B, dtype=jnp.int32) * N)[:, None, None]
    idx = (indices_.astype(jnp.int32) + offs).reshape(-1)

    next_pad = jnp.pad(next_pts.reshape(B * N, DIM), ((0, 0), (0, 16 - DIM)))

    a1 = w1.reshape(2 * NC, DIM, NC).sum(axis=2).T
    a1_pad = jnp.pad(a1, ((0, 16 - DIM), (0, 0)))
    c1 = b1 - w1 @ centers.reshape(DIM * NC)
    wflat = weight.transpose(1, 0, 2).reshape(NC * C_IN, C_OUT) / KK
    rep = (jnp.arange(256)[None, :] // 16 ==
           jnp.arange(16)[:, None]).astype(jnp.bfloat16)

    gathered = _sc_gather(table, idx)
    out = _tc_compute(gathered, next_pad, a1_pad, c1, w2.T, b2, w3.T, b3,
                      wflat, bias, rep, mt=256, kk=KK, c_in=C_IN, c_out=C_OUT)
    return out.reshape(B, N, C_OUT)


# speedup vs baseline: 3.1580x; 1.1461x over previous
import functools

import jax
import jax.numpy as jnp
from jax import lax
from jax.experimental import pallas as pl
from jax.experimental.pallas import tpu as pltpu
from jax.experimental.pallas import tpu_sc as plsc


def _sc_gather(table, idx):
    V, D = table.shape
    Btot = idx.shape[0]
    info = plsc.get_sparse_core_info()
    NC_, NS, L = info.num_cores, info.num_subcores, info.num_lanes
    NW = NC_ * NS
    assert D % L == 0 and Btot % (8 * NW) == 0
    b_per_w = Btot // NW
    CH = 256
    while b_per_w % CH != 0:
        CH //= 2
    n_chunks = b_per_w // CH
    mesh = plsc.VectorSubcoreMesh(core_axis_name="c", subcore_axis_name="s")

    @functools.partial(
        pl.kernel, mesh=mesh,
        compiler_params=pltpu.CompilerParams(use_tc_tiling_on_sc=False),
        out_type=jax.ShapeDtypeStruct((Btot, D), table.dtype),
        scratch_types=[
            pltpu.VMEM((b_per_w,), jnp.int32),
            pltpu.VMEM((CH, D), table.dtype),
            pltpu.VMEM((CH, D), table.dtype),
            pltpu.SemaphoreType.DMA,
            pltpu.SemaphoreType.DMA,
            pltpu.SemaphoreType.DMA,
            pltpu.SemaphoreType.DMA,
        ],
    )
    def k(table_hbm, idx_hbm, out_hbm, idx_v, buf0, buf1, sg0, sg1, sw0, sw1):
        wid = lax.axis_index("s") * NC_ + lax.axis_index("c")
        base = wid * b_per_w
        pltpu.sync_copy(idx_hbm.at[pl.ds(base, b_per_w)], idx_v)
        bufs = (buf0, buf1)
        gsems = (sg0, sg1)
        wsems = (sw0, sw1)

        def gather_start(g, slot):
            pltpu.async_copy(
                table_hbm.at[idx_v.at[pl.ds(g * CH, CH)]], bufs[slot], gsems[slot])

        def write_start(g, slot):
            pltpu.async_copy(
                bufs[slot], out_hbm.at[pl.ds(base + g * CH, CH)], wsems[slot])

        gather_start(0, 0)

        def body(i, carry):
            for b in (0, 1):
                g = 2 * i + b
                nslot = 1 - b
                pltpu.make_async_copy(
                    table_hbm.at[idx_v.at[pl.ds(g * CH, CH)]], bufs[b], gsems[b]
                ).wait()

                @pl.when(g + 1 < n_chunks)
                def _():
                    @pl.when(g >= 1)
                    def _():
                        pltpu.make_async_copy(
                            bufs[nslot],
                            out_hbm.at[pl.ds(base + (g - 1) * CH, CH)],
                            wsems[nslot],
                        ).wait()
                    gather_start(g + 1, nslot)

                write_start(g, b)
            return carry

        lax.fori_loop(0, n_chunks // 2, body, 0)
        pltpu.make_async_copy(
            bufs[(n_chunks - 1) % 2],
            out_hbm.at[pl.ds(base + (n_chunks - 1) * CH, CH)],
            wsems[(n_chunks - 1) % 2],
        ).wait()

        @pl.when(n_chunks >= 2)
        def _():
            pltpu.make_async_copy(
                bufs[(n_chunks - 2) % 2],
                out_hbm.at[pl.ds(base + (n_chunks - 2) * CH, CH)],
                wsems[(n_chunks - 2) % 2],
            ).wait()

    return k(table, idx)


def _tc_body(g_ref, next_ref, a1_ref, c1_ref, w2_ref, b2_ref, w3_ref, b3_ref,
             wf_ref, bias_ref, rep_ref, out_ref, v_ref, *, mt, kk, c_in):
    G = g_ref[...]
    feat = G[:, :c_in]
    ptsp = G[:, c_in:c_in + 16]

    a1 = a1_ref[...]
    r = jnp.dot(next_ref[...], a1, preferred_element_type=jnp.float32)
    h1 = jnp.dot(ptsp, a1, preferred_element_type=jnp.float32)
    h1 = h1.reshape(mt, kk, 32) - r[:, None, :] + c1_ref[...][None, None, :]
    h1 = jnp.maximum(h1.reshape(mt * kk, 32), 0.0)
    h2 = jnp.maximum(
        jnp.dot(h1, w2_ref[...], preferred_element_type=jnp.float32)
        + b2_ref[...][None, :], 0.0)
    d = jnp.maximum(
        jnp.dot(h2, w3_ref[...], preferred_element_type=jnp.float32)
        + b3_ref[...][None, :], 0.0)

    PG = 16
    rows = mt * kk
    bd_w = PG * 16
    dt = jnp.dot(d.astype(jnp.bfloat16), rep_ref[...],
                 preferred_element_type=jnp.float32)
    ri = lax.broadcasted_iota(jnp.int32, (rows, bd_w), 0)
    li = lax.broadcasted_iota(jnp.int32, (rows, bd_w), 1)
    msk = (li % PG) == ((ri // kk) % PG)
    dbd = jnp.where(msk, dt, 0.0).astype(jnp.bfloat16)
    featb = feat.astype(jnp.bfloat16)
    grp = PG * kk
    for g in range(mt // PG):
        dg = lax.slice(dbd, (g * grp, 0), ((g + 1) * grp, bd_w))
        fg = lax.slice(featb, (g * grp, 0), ((g + 1) * grp, c_in))
        vg = lax.dot_general(
            dg, fg, (((0,), (0,)), ((), ())),
            preferred_element_type=jnp.float32)
        for n in range(16):
            v_ref[pl.ds(g * PG, PG), pl.ds(n * c_in, c_in)] = (
                lax.slice(vg, (n * PG, 0), ((n + 1) * PG, c_in)))
    out_ref[...] = (
        jnp.dot(v_ref[...], wf_ref[...], preferred_element_type=jnp.float32)
        + bias_ref[...][None, :])


def _tc_compute(gathered, next_pad, a1_pad, c1, w2t, b2, w3t, b3, wflat, bias,
                rep, mt, kk, c_in, c_out):
    G = gathered.shape[0] // kk
    n_tiles = G // mt
    grid = (n_tiles,)
    body = functools.partial(_tc_body, mt=mt, kk=kk, c_in=c_in)
    return pl.pallas_call(
        body,
        grid=grid,
        in_specs=[
            pl.BlockSpec((mt * kk, 128), lambda i: (i, 0)),
            pl.BlockSpec((mt, 16), lambda i: (i, 0)),
            pl.BlockSpec((16, 32), lambda i: (0, 0)),
            pl.BlockSpec((32,), lambda i: (0,)),
            pl.BlockSpec((32, 16), lambda i: (0, 0)),
            pl.BlockSpec((16,), lambda i: (0,)),
            pl.BlockSpec((16, 16), lambda i: (0, 0)),
            pl.BlockSpec((16,), lambda i: (0,)),
            pl.BlockSpec((16 * c_in, c_out), lambda i: (0, 0)),
            pl.BlockSpec((c_out,), lambda i: (0,)),
            pl.BlockSpec((16, 256), lambda i: (0, 0)),
        ],
        out_specs=pl.BlockSpec((mt, c_out), lambda i: (i, 0)),
        out_shape=jax.ShapeDtypeStruct((G, c_out), jnp.float32),
        scratch_shapes=[pltpu.VMEM((mt, 16 * c_in), jnp.float32)],
    )(gathered, next_pad, a1_pad, c1, w2t, b2, w3t, b3, wflat, bias, rep)


def kernel(inp, points, next_pts, indices_, K, weight, bias, centers,
           w1, b1, w2, b2, w3, b3):
    B, N, C_IN = inp.shape
    KK = indices_.shape[2]
    DIM = points.shape[2]
    NC = centers.shape[1]
    C_OUT = weight.shape[2]

    inp_flat = inp.reshape(B * N, C_IN)
    pts_pad = jnp.pad(points.reshape(B * N, DIM), ((0, 0), (0, 16 - DIM)))
    table = jnp.concatenate([inp_flat, pts_pad], axis=1)
    table = jnp.pad(table, ((0, 0), (0, 48)))
    offs = (jnp.arange(B, dtype=jnp.int32) * N)[:, None, None]
    idx = (indices_.astype(jnp.int32) + offs).reshape(-1)

    next_pad = jnp.pad(next_pts.reshape(B * N, DIM), ((0, 0), (0, 16 - DIM)))

    a1 = w1.reshape(2 * NC, DIM, NC).sum(axis=2).T
    a1_pad = jnp.pad(a1, ((0, 16 - DIM), (0, 0)))
    c1 = b1 - w1 @ centers.reshape(DIM * NC)
    wflat = weight.transpose(1, 0, 2).reshape(NC * C_IN, C_OUT) / KK
    rep = (jnp.arange(256)[None, :] // 16 ==
           jnp.arange(16)[:, None]).astype(jnp.bfloat16)

    NCHUNK = 4
    pts_per_chunk = (B * N) // NCHUNK
    rows_per_chunk = pts_per_chunk * KK
    outs = []
    for c in range(NCHUNK):
        idx_c = lax.slice(idx, (c * rows_per_chunk,), ((c + 1) * rows_per_chunk,))
        next_c = lax.slice(next_pad, (c * pts_per_chunk, 0),
                           ((c + 1) * pts_per_chunk, 16))
        gath_c = _sc_gather(table, idx_c)
        outs.append(_tc_compute(gath_c, next_c, a1_pad, c1, w2.T, b2, w3.T, b3,
                                wflat, bias, rep, mt=256, kk=KK, c_in=C_IN,
                                c_out=C_OUT))
    out = jnp.concatenate(outs, axis=0)
    return out.reshape(B, N, C_OUT)


# speedup vs baseline: 3.2098x; 1.0164x over previous
import functools

import jax
import jax.numpy as jnp
from jax import lax
from jax.experimental import pallas as pl
from jax.experimental.pallas import tpu as pltpu
from jax.experimental.pallas import tpu_sc as plsc


def _sc_gather(table, idx):
    V, D = table.shape
    Btot = idx.shape[0]
    info = plsc.get_sparse_core_info()
    NC_, NS, L = info.num_cores, info.num_subcores, info.num_lanes
    NW = NC_ * NS
    assert D % L == 0 and Btot % (8 * NW) == 0
    b_per_w = Btot // NW
    CH = 256
    while b_per_w % CH != 0:
        CH //= 2
    n_chunks = b_per_w // CH
    mesh = plsc.VectorSubcoreMesh(core_axis_name="c", subcore_axis_name="s")

    @functools.partial(
        pl.kernel, mesh=mesh,
        compiler_params=pltpu.CompilerParams(use_tc_tiling_on_sc=False),
        out_type=jax.ShapeDtypeStruct((Btot, D), table.dtype),
        scratch_types=[
            pltpu.VMEM((b_per_w,), jnp.int32),
            pltpu.VMEM((CH, D), table.dtype),
            pltpu.VMEM((CH, D), table.dtype),
            pltpu.SemaphoreType.DMA,
            pltpu.SemaphoreType.DMA,
            pltpu.SemaphoreType.DMA,
            pltpu.SemaphoreType.DMA,
        ],
    )
    def k(table_hbm, idx_hbm, out_hbm, idx_v, buf0, buf1, sg0, sg1, sw0, sw1):
        wid = lax.axis_index("s") * NC_ + lax.axis_index("c")
        base = wid * b_per_w
        pltpu.sync_copy(idx_hbm.at[pl.ds(base, b_per_w)], idx_v)
        bufs = (buf0, buf1)
        gsems = (sg0, sg1)
        wsems = (sw0, sw1)

        def gather_start(g, slot):
            pltpu.async_copy(
                table_hbm.at[idx_v.at[pl.ds(g * CH, CH)]], bufs[slot], gsems[slot])

        def write_start(g, slot):
            pltpu.async_copy(
                bufs[slot], out_hbm.at[pl.ds(base + g * CH, CH)], wsems[slot])

        gather_start(0, 0)

        def body(i, carry):
            for b in (0, 1):
                g = 2 * i + b
                nslot = 1 - b
                pltpu.make_async_copy(
                    table_hbm.at[idx_v.at[pl.ds(g * CH, CH)]], bufs[b], gsems[b]
                ).wait()

                @pl.when(g + 1 < n_chunks)
                def _():
                    @pl.when(g >= 1)
                    def _():
                        pltpu.make_async_copy(
                            bufs[nslot],
                            out_hbm.at[pl.ds(base + (g - 1) * CH, CH)],
                            wsems[nslot],
                        ).wait()
                    gather_start(g + 1, nslot)

                write_start(g, b)
            return carry

        lax.fori_loop(0, n_chunks // 2, body, 0)
        pltpu.make_async_copy(
            bufs[(n_chunks - 1) % 2],
            out_hbm.at[pl.ds(base + (n_chunks - 1) * CH, CH)],
            wsems[(n_chunks - 1) % 2],
        ).wait()

        @pl.when(n_chunks >= 2)
        def _():
            pltpu.make_async_copy(
                bufs[(n_chunks - 2) % 2],
                out_hbm.at[pl.ds(base + (n_chunks - 2) * CH, CH)],
                wsems[(n_chunks - 2) % 2],
            ).wait()

    return k(table, idx)


def _prep_body(inp_ref, pts_ref, ind_ref, tab_ref, idx_ref, *, n, kk):
    r = inp_ref.shape[0]
    tab_ref[...] = jnp.concatenate(
        [inp_ref[...], pts_ref[...],
         jnp.zeros((r, 128 - inp_ref.shape[1] - pts_ref.shape[1]),
                   jnp.float32)], axis=1)
    gi = pl.program_id(0)
    ir = ind_ref.shape[0]
    e = ((gi * ir + lax.broadcasted_iota(jnp.int32, (ir, 128), 0)) * 128
         + lax.broadcasted_iota(jnp.int32, (ir, 128), 1))
    idx_ref[...] = ind_ref[...] + (e // (n * kk)) * n


def _prep(inp_flat, pts_flat, ind2, n, kk):
    R = inp_flat.shape[0]
    steps = 8
    rb = R // steps
    ib = ind2.shape[0] // steps
    body = functools.partial(_prep_body, n=n, kk=kk)
    return pl.pallas_call(
        body,
        grid=(steps,),
        in_specs=[
            pl.BlockSpec((rb, inp_flat.shape[1]), lambda i: (i, 0)),
            pl.BlockSpec((rb, pts_flat.shape[1]), lambda i: (i, 0)),
            pl.BlockSpec((ib, 128), lambda i: (i, 0)),
        ],
        out_specs=[
            pl.BlockSpec((rb, 128), lambda i: (i, 0)),
            pl.BlockSpec((ib, 128), lambda i: (i, 0)),
        ],
        out_shape=[
            jax.ShapeDtypeStruct((R, 128), jnp.float32),
            jax.ShapeDtypeStruct((ind2.shape[0], 128), jnp.int32),
        ],
    )(inp_flat, pts_flat, ind2)


def _tc_body(g_ref, next_ref, a1_ref, c1_ref, w2_ref, b2_ref, w3_ref, b3_ref,
             wf_ref, bias_ref, rep_ref, out_ref, v_ref, *, mt, kk, c_in):
    G = g_ref[...]
    feat = G[:, :c_in]
    ptsp = G[:, c_in:c_in + 16]

    a1 = a1_ref[...]
    r = jnp.dot(next_ref[...].astype(jnp.bfloat16), a1,
                preferred_element_type=jnp.float32)
    h1 = jnp.dot(ptsp.astype(jnp.bfloat16), a1,
                 preferred_element_type=jnp.float32)
    h1 = h1.reshape(mt, kk, 32) - r[:, None, :] + c1_ref[...][None, None, :]
    h1 = jnp.maximum(h1.reshape(mt * kk, 32), 0.0)
    h2 = jnp.maximum(
        jnp.dot(h1, w2_ref[...], preferred_element_type=jnp.float32)
        + b2_ref[...][None, :], 0.0)
    d = jnp.maximum(
        jnp.dot(h2, w3_ref[...], preferred_element_type=jnp.float32)
        + b3_ref[...][None, :], 0.0)

    PG = 16
    rows = mt * kk
    bd_w = PG * 16
    dt = jnp.dot(d.astype(jnp.bfloat16), rep_ref[...],
                 preferred_element_type=jnp.float32)
    ri = lax.broadcasted_iota(jnp.int32, (rows, bd_w), 0)
    li = lax.broadcasted_iota(jnp.int32, (rows, bd_w), 1)
    msk = (li % PG) == ((ri // kk) % PG)
    dbd = jnp.where(msk, dt, 0.0).astype(jnp.bfloat16)
    featb = feat.astype(jnp.bfloat16)
    grp = PG * kk
    for g in range(mt // PG):
        dg = lax.slice(dbd, (g * grp, 0), ((g + 1) * grp, bd_w))
        fg = lax.slice(featb, (g * grp, 0), ((g + 1) * grp, c_in))
        vg = lax.dot_general(
            dg, fg, (((0,), (0,)), ((), ())),
            preferred_element_type=jnp.float32)
        for n in range(16):
            v_ref[pl.ds(g * PG, PG), pl.ds(n * c_in, c_in)] = (
                lax.slice(vg, (n * PG, 0), ((n + 1) * PG, c_in)))
    out_ref[...] = (
        jnp.dot(v_ref[...].astype(jnp.bfloat16), wf_ref[...],
                preferred_element_type=jnp.float32)
        + bias_ref[...][None, :])


def _tc_compute(gathered, next_pad, a1_pad, c1, w2t, b2, w3t, b3, wflat, bias,
                rep, mt, kk, c_in, c_out):
    G = gathered.shape[0] // kk
    n_tiles = G // mt
    grid = (n_tiles,)
    body = functools.partial(_tc_body, mt=mt, kk=kk, c_in=c_in)
    return pl.pallas_call(
        body,
        grid=grid,
        in_specs=[
            pl.BlockSpec((mt * kk, 128), lambda i: (i, 0)),
            pl.BlockSpec((mt, 16), lambda i: (i, 0)),
            pl.BlockSpec((16, 32), lambda i: (0, 0)),
            pl.BlockSpec((32,), lambda i: (0,)),
            pl.BlockSpec((32, 16), lambda i: (0, 0)),
            pl.BlockSpec((16,), lambda i: (0,)),
            pl.BlockSpec((16, 16), lambda i: (0, 0)),
            pl.BlockSpec((16,), lambda i: (0,)),
            pl.BlockSpec((16 * c_in, c_out), lambda i: (0, 0)),
            pl.BlockSpec((c_out,), lambda i: (0,)),
            pl.BlockSpec((16, 256), lambda i: (0, 0)),
        ],
        out_specs=pl.BlockSpec((mt, c_out), lambda i: (i, 0)),
        out_shape=jax.ShapeDtypeStruct((G, c_out), jnp.float32),
        scratch_shapes=[pltpu.VMEM((mt, 16 * c_in), jnp.float32)],
    )(gathered, next_pad, a1_pad, c1, w2t, b2, w3t, b3, wflat, bias, rep)


def kernel(inp, points, next_pts, indices_, K, weight, bias, centers,
           w1, b1, w2, b2, w3, b3):
    B, N, C_IN = inp.shape
    KK = indices_.shape[2]
    DIM = points.shape[2]
    NC = centers.shape[1]
    C_OUT = weight.shape[2]

    inp_flat = inp.reshape(B * N, C_IN)
    pts_flat = points.reshape(B * N, DIM)
    ind2 = indices_.astype(jnp.int32).reshape(B * N * KK // 128, 128)
    table, idx2 = _prep(inp_flat, pts_flat, ind2, N, KK)
    idx = idx2.reshape(-1)

    next_pad = jnp.pad(next_pts.reshape(B * N, DIM), ((0, 0), (0, 16 - DIM)))

    a1 = w1.reshape(2 * NC, DIM, NC).sum(axis=2).T
    a1_pad = jnp.pad(a1, ((0, 16 - DIM), (0, 0)))
    c1 = b1 - w1 @ centers.reshape(DIM * NC)
    wflat = weight.transpose(1, 0, 2).reshape(NC * C_IN, C_OUT) / KK
    rep = (jnp.arange(256)[None, :] // 16 ==
           jnp.arange(16)[:, None]).astype(jnp.bfloat16)

    NCHUNK = 4
    pts_per_chunk = (B * N) // NCHUNK
    rows_per_chunk = pts_per_chunk * KK
    outs = []
    for c in range(NCHUNK):
        idx_c = lax.slice(idx, (c * rows_per_chunk,), ((c + 1) * rows_per_chunk,))
        next_c = lax.slice(next_pad, (c * pts_per_chunk, 0),
                           ((c + 1) * pts_per_chunk, 16))
        gath_c = _sc_gather(table, idx_c)
        outs.append(_tc_compute(gath_c, next_c, a1_pad.astype(jnp.bfloat16),
                                c1, w2.T, b2, w3.T, b3,
                                wflat.astype(jnp.bfloat16), bias, rep,
                                mt=256, kk=KK, c_in=C_IN, c_out=C_OUT))
    out = jnp.concatenate(outs, axis=0)
    return out.reshape(B, N, C_OUT)


# speedup vs baseline: 3.2525x; 1.0133x over previous
import functools

import jax
import jax.numpy as jnp
from jax import lax
from jax.experimental import pallas as pl
from jax.experimental.pallas import tpu as pltpu
from jax.experimental.pallas import tpu_sc as plsc


def _sc_gather(table, idx):
    V, D = table.shape
    Btot = idx.shape[0]
    info = plsc.get_sparse_core_info()
    NC_, NS, L = info.num_cores, info.num_subcores, info.num_lanes
    NW = NC_ * NS
    assert D % L == 0 and Btot % (8 * NW) == 0
    b_per_w = Btot // NW
    CH = 256
    while b_per_w % CH != 0:
        CH //= 2
    n_chunks = b_per_w // CH
    mesh = plsc.VectorSubcoreMesh(core_axis_name="c", subcore_axis_name="s")

    @functools.partial(
        pl.kernel, mesh=mesh,
        compiler_params=pltpu.CompilerParams(use_tc_tiling_on_sc=False),
        out_type=jax.ShapeDtypeStruct((Btot, D), table.dtype),
        scratch_types=[
            pltpu.VMEM((b_per_w,), jnp.int32),
            pltpu.VMEM((CH, D), table.dtype),
            pltpu.VMEM((CH, D), table.dtype),
            pltpu.SemaphoreType.DMA,
            pltpu.SemaphoreType.DMA,
            pltpu.SemaphoreType.DMA,
            pltpu.SemaphoreType.DMA,
        ],
    )
    def k(table_hbm, idx_hbm, out_hbm, idx_v, buf0, buf1, sg0, sg1, sw0, sw1):
        wid = lax.axis_index("s") * NC_ + lax.axis_index("c")
        base = wid * b_per_w
        pltpu.sync_copy(idx_hbm.at[pl.ds(base, b_per_w)], idx_v)
        bufs = (buf0, buf1)
        gsems = (sg0, sg1)
        wsems = (sw0, sw1)

        def gather_start(g, slot):
            pltpu.async_copy(
                table_hbm.at[idx_v.at[pl.ds(g * CH, CH)]], bufs[slot], gsems[slot])

        def write_start(g, slot):
            pltpu.async_copy(
                bufs[slot], out_hbm.at[pl.ds(base + g * CH, CH)], wsems[slot])

        gather_start(0, 0)

        def body(i, carry):
            for b in (0, 1):
                g = 2 * i + b
                nslot = 1 - b
                pltpu.make_async_copy(
                    table_hbm.at[idx_v.at[pl.ds(g * CH, CH)]], bufs[b], gsems[b]
                ).wait()

                @pl.when(g + 1 < n_chunks)
                def _():
                    @pl.when(g >= 1)
                    def _():
                        pltpu.make_async_copy(
                            bufs[nslot],
                            out_hbm.at[pl.ds(base + (g - 1) * CH, CH)],
                            wsems[nslot],
                        ).wait()
                    gather_start(g + 1, nslot)

                write_start(g, b)
            return carry

        lax.fori_loop(0, n_chunks // 2, body, 0)
        pltpu.make_async_copy(
            bufs[(n_chunks - 1) % 2],
            out_hbm.at[pl.ds(base + (n_chunks - 1) * CH, CH)],
            wsems[(n_chunks - 1) % 2],
        ).wait()

        @pl.when(n_chunks >= 2)
        def _():
            pltpu.make_async_copy(
                bufs[(n_chunks - 2) % 2],
                out_hbm.at[pl.ds(base + (n_chunks - 2) * CH, CH)],
                wsems[(n_chunks - 2) % 2],
            ).wait()

    return k(table, idx)


def _prep_body(inp_ref, pts_ref, ind_ref, tab_ref, idx_ref, *, n, kk):
    r = inp_ref.shape[0]
    tab_ref[...] = jnp.concatenate(
        [inp_ref[...], pts_ref[...],
         jnp.zeros((r, 128 - inp_ref.shape[1] - pts_ref.shape[1]),
                   jnp.float32)], axis=1)
    gi = pl.program_id(0)
    ir = ind_ref.shape[0]
    e = ((gi * ir + lax.broadcasted_iota(jnp.int32, (ir, 128), 0)) * 128
         + lax.broadcasted_iota(jnp.int32, (ir, 128), 1))
    idx_ref[...] = ind_ref[...] + (e // (n * kk)) * n


def _prep(inp_flat, pts_flat, ind2, n, kk):
    R = inp_flat.shape[0]
    steps = 8
    rb = R // steps
    ib = ind2.shape[0] // steps
    body = functools.partial(_prep_body, n=n, kk=kk)
    return pl.pallas_call(
        body,
        grid=(steps,),
        in_specs=[
            pl.BlockSpec((rb, inp_flat.shape[1]), lambda i: (i, 0)),
            pl.BlockSpec((rb, pts_flat.shape[1]), lambda i: (i, 0)),
            pl.BlockSpec((ib, 128), lambda i: (i, 0)),
        ],
        out_specs=[
            pl.BlockSpec((rb, 128), lambda i: (i, 0)),
            pl.BlockSpec((ib, 128), lambda i: (i, 0)),
        ],
        out_shape=[
            jax.ShapeDtypeStruct((R, 128), jnp.float32),
            jax.ShapeDtypeStruct((ind2.shape[0], 128), jnp.int32),
        ],
    )(inp_flat, pts_flat, ind2)


def _tc_body(g_ref, next_ref, a1_ref, c1_ref, w2_ref, b2_ref, w3_ref, b3_ref,
             wf_ref, bias_ref, rep_ref, out_ref, v_ref, *, mt, kk, c_in):
    G = g_ref[...]
    feat = G[:, :c_in]
    ptsp = G[:, c_in:c_in + 16]

    a1 = a1_ref[...]
    r = jnp.dot(next_ref[...].astype(jnp.bfloat16), a1,
                preferred_element_type=jnp.float32)
    h1 = jnp.dot(ptsp.astype(jnp.bfloat16), a1,
                 preferred_element_type=jnp.float32)
    h1 = h1.reshape(mt, kk, 32) - r[:, None, :] + c1_ref[...][None, None, :]
    h1 = jnp.maximum(h1.reshape(mt * kk, 32), 0.0)
    h2 = jnp.maximum(
        jnp.dot(h1, w2_ref[...], preferred_element_type=jnp.float32)
        + b2_ref[...][None, :], 0.0)
    d = jnp.maximum(
        jnp.dot(h2, w3_ref[...], preferred_element_type=jnp.float32)
        + b3_ref[...][None, :], 0.0)

    PG = 16
    rows = mt * kk
    bd_w = PG * 16
    dt = jnp.dot(d.astype(jnp.bfloat16), rep_ref[...],
                 preferred_element_type=jnp.float32)
    ri = lax.broadcasted_iota(jnp.int32, (rows, bd_w), 0)
    li = lax.broadcasted_iota(jnp.int32, (rows, bd_w), 1)
    msk = (li % PG) == ((ri // kk) % PG)
    dbd = jnp.where(msk, dt, 0.0).astype(jnp.bfloat16)
    featb = feat.astype(jnp.bfloat16)
    grp = PG * kk
    for g in range(mt // PG):
        dg = lax.slice(dbd, (g * grp, 0), ((g + 1) * grp, bd_w))
        fg = lax.slice(featb, (g * grp, 0), ((g + 1) * grp, c_in))
        vg = lax.dot_general(
            dg, fg, (((0,), (0,)), ((), ())),
            preferred_element_type=jnp.float32)
        for n in range(16):
            v_ref[pl.ds(g * PG, PG), pl.ds(n * c_in, c_in)] = (
                lax.slice(vg, (n * PG, 0), ((n + 1) * PG, c_in)))
    out_ref[...] = (
        jnp.dot(v_ref[...].astype(jnp.bfloat16), wf_ref[...],
                preferred_element_type=jnp.float32)
        + bias_ref[...][None, :])


def _tc_compute(gathered, next_pad, a1_pad, c1, w2t, b2, w3t, b3, wflat, bias,
                rep, mt, kk, c_in, c_out):
    G = gathered.shape[0] // kk
    n_tiles = G // mt
    grid = (n_tiles,)
    body = functools.partial(_tc_body, mt=mt, kk=kk, c_in=c_in)
    return pl.pallas_call(
        body,
        grid=grid,
        in_specs=[
            pl.BlockSpec((mt * kk, 128), lambda i: (i, 0)),
            pl.BlockSpec((mt, 16), lambda i: (i, 0)),
            pl.BlockSpec((16, 32), lambda i: (0, 0)),
            pl.BlockSpec((32,), lambda i: (0,)),
            pl.BlockSpec((32, 16), lambda i: (0, 0)),
            pl.BlockSpec((16,), lambda i: (0,)),
            pl.BlockSpec((16, 16), lambda i: (0, 0)),
            pl.BlockSpec((16,), lambda i: (0,)),
            pl.BlockSpec((16 * c_in, c_out), lambda i: (0, 0)),
            pl.BlockSpec((c_out,), lambda i: (0,)),
            pl.BlockSpec((16, 256), lambda i: (0, 0)),
        ],
        out_specs=pl.BlockSpec((mt, c_out), lambda i: (i, 0)),
        out_shape=jax.ShapeDtypeStruct((G, c_out), jnp.float32),
        scratch_shapes=[pltpu.VMEM((mt, 16 * c_in), jnp.float32)],
    )(gathered, next_pad, a1_pad, c1, w2t, b2, w3t, b3, wflat, bias, rep)


def kernel(inp, points, next_pts, indices_, K, weight, bias, centers,
           w1, b1, w2, b2, w3, b3):
    B, N, C_IN = inp.shape
    KK = indices_.shape[2]
    DIM = points.shape[2]
    NC = centers.shape[1]
    C_OUT = weight.shape[2]

    inp_flat = inp.reshape(B * N, C_IN)
    pts_flat = points.reshape(B * N, DIM)
    ind2 = indices_.astype(jnp.int32).reshape(B * N * KK // 128, 128)
    table, idx2 = _prep(inp_flat, pts_flat, ind2, N, KK)
    idx = idx2.reshape(-1)

    next_pad = jnp.pad(next_pts.reshape(B * N, DIM), ((0, 0), (0, 16 - DIM)))

    a1 = w1.reshape(2 * NC, DIM, NC).sum(axis=2).T
    a1_pad = jnp.pad(a1, ((0, 16 - DIM), (0, 0)))
    c1 = b1 - w1 @ centers.reshape(DIM * NC)
    wflat = weight.transpose(1, 0, 2).reshape(NC * C_IN, C_OUT) / KK
    rep = (jnp.arange(256)[None, :] // 16 ==
           jnp.arange(16)[:, None]).astype(jnp.bfloat16)

    NCHUNK = 8
    pts_per_chunk = (B * N) // NCHUNK
    rows_per_chunk = pts_per_chunk * KK
    outs = []
    for c in range(NCHUNK):
        idx_c = lax.slice(idx, (c * rows_per_chunk,), ((c + 1) * rows_per_chunk,))
        next_c = lax.slice(next_pad, (c * pts_per_chunk, 0),
                           ((c + 1) * pts_per_chunk, 16))
        gath_c = _sc_gather(table, idx_c)
        outs.append(_tc_compute(gath_c, next_c, a1_pad.astype(jnp.bfloat16),
                                c1, w2.T, b2, w3.T, b3,
                                wflat.astype(jnp.bfloat16), bias, rep,
                                mt=256, kk=KK, c_in=C_IN, c_out=C_OUT))
    out = jnp.concatenate(outs, axis=0)
    return out.reshape(B, N, C_OUT)
